# baseline (speedup 1.0000x reference)
import functools

import jax
import jax.numpy as jnp
from jax import lax
from jax.experimental import pallas as pl
from jax.experimental.pallas import tpu as pltpu
from jax.experimental.pallas import tpu_sc as plsc

LOSS_WEIGHT = 1.0
NEG_FILL = -40.0

N_ROWS = 100000
N_COLS = 80

_NC, _NS, _L = 2, 16, 16
_NW = _NC * _NS
_PER_W = 4096
_NPAD = _NW * _PER_W
_RCHUNK = 256
_NCHUNK = _PER_W // _RCHUNK


_LN2 = 0.6931471805599453


def _fsum_scaled(x):
    th = jnp.tanh(0.5 * x)
    h = 0.5 * th
    sig = 0.5 + h
    m = 0.5 - h
    return jnp.sum(sig * sig * jnp.log2(m), axis=(0, 1), keepdims=True)


def _dense_sum_kernel(x_ref, out_ref):
    i = pl.program_id(0)
    partial = _fsum_scaled(x_ref[...])

    @pl.when(i == 0)
    def _():
        out_ref[...] = jnp.zeros_like(out_ref)

    out_ref[...] += partial


def _small_sum_kernel(g_ref, out_ref):
    out_ref[...] = _fsum_scaled(g_ref[...])


def _sc_gather_kernel(x_hbm, lbl_hbm, out_hbm, lbl_v, rows_v, g_v, sem0, sem1):
    wid = lax.axis_index("s") * _NC + lax.axis_index("c")
    base = wid * _PER_W
    pltpu.sync_copy(lbl_hbm.at[pl.ds(base, _PER_W)], lbl_v)

    sems = [sem0, sem1]
    iota = lax.iota(jnp.int32, _L)

    def dma_base(chunk):
        return jnp.minimum(base + chunk * _RCHUNK, N_ROWS - _RCHUNK)

    pltpu.make_async_copy(
        x_hbm.at[pl.ds(dma_base(0), _RCHUNK)], rows_v.at[0], sems[0]
    ).start()

    for chunk in range(_NCHUNK):
        b = chunk % 2
        pltpu.make_async_copy(
            x_hbm.at[pl.ds(dma_base(chunk), _RCHUNK)], rows_v.at[b], sems[b]
        ).wait()
        if chunk + 1 < _NCHUNK:
            pltpu.make_async_copy(
                x_hbm.at[pl.ds(dma_base(chunk + 1), _RCHUNK)],
                rows_v.at[1 - b],
                sems[1 - b],
            ).start()

        chunk_base = base + chunk * _RCHUNK
        dbase = dma_base(chunk)

        def body(i, _, chunk=chunk, chunk_base=chunk_base, dbase=dbase, b=b):
            l = lbl_v[pl.ds(chunk * _RCHUNK + i * _L, _L)]
            r = chunk_base + i * _L + iota
            valid = (l < N_COLS) & (r < N_ROWS)
            l_safe = jnp.where(valid, l, 0)
            local = jnp.minimum(r, N_ROWS - 1) - dbase
            got = plsc.load_gather(rows_v.at[b], [local, l_safe])
            g_v[chunk, pl.ds(i * _L, _L)] = jnp.where(valid, got, NEG_FILL)
            return _

        lax.fori_loop(0, _RCHUNK // _L, body, None)

    pltpu.sync_copy(g_v, out_hbm.at[pl.ds(wid * _NCHUNK, _NCHUNK)])


def kernel(cls_pred, cls_label):
    N, C = cls_pred.shape

    s1 = pl.pallas_call(
        _dense_sum_kernel,
        grid=(10,),
        in_specs=[pl.BlockSpec((N // 10, C), lambda i: (i, 0))],
        out_specs=pl.BlockSpec((1, 1), lambda i: (0, 0)),
        out_shape=jax.ShapeDtypeStruct((1, 1), jnp.float32),
    )(cls_pred)

    lbl = cls_label.astype(jnp.int32)
    lblp = jnp.concatenate([lbl, jnp.full((_NPAD - N,), C, jnp.int32)])

    mesh = plsc.VectorSubcoreMesh(core_axis_name="c", subcore_axis_name="s")
    g = pl.kernel(
        _sc_gather_kernel,
        mesh=mesh,
        out_type=jax.ShapeDtypeStruct((_NW * _NCHUNK, _RCHUNK), jnp.float32),
        scratch_types=[
            pltpu.VMEM((_PER_W,), jnp.int32),
            pltpu.VMEM((2, _RCHUNK, N_COLS), jnp.float32),
            pltpu.VMEM((_NCHUNK, _RCHUNK), jnp.float32),
            pltpu.SemaphoreType.DMA,
            pltpu.SemaphoreType.DMA,
        ],
        compiler_params=pltpu.CompilerParams(
            use_tc_tiling_on_sc=True, needs_layout_passes=False
        ),
    )(cls_pred, lblp)

    s2 = pl.pallas_call(
        _small_sum_kernel,
        grid=(1,),
        in_specs=[pl.BlockSpec(g.shape, lambda i: (0, 0))],
        out_specs=pl.BlockSpec((1, 1), lambda i: (0, 0)),
        out_shape=jax.ShapeDtypeStruct((1, 1), jnp.float32),
    )(g)

    return (-_LN2 * LOSS_WEIGHT / (N * C)) * (0.75 * s1[0, 0] - 0.5 * s2[0, 0])


# speedup vs baseline: 1.2158x; 1.2158x over previous
import functools

import jax
import jax.numpy as jnp
from jax import lax
from jax.experimental import pallas as pl
from jax.experimental.pallas import tpu as pltpu
from jax.experimental.pallas import tpu_sc as plsc

LOSS_WEIGHT = 1.0
NEG_FILL = -40.0

N_ROWS = 100000
N_COLS = 80

_NC, _NS, _L = 2, 16, 16
_NW = _NC * _NS
_PER_W = 3200
_NPAD = _NW * _PER_W
_RCHUNK = 400
_NCHUNK = _PER_W // _RCHUNK

_LN2 = 0.6931471805599453


def _fsum_scaled(x):
    th = jnp.tanh(0.5 * x)
    h = 0.5 * th
    sig = 0.5 + h
    m = 0.5 - h
    return jnp.sum(sig * sig * jnp.log2(m), axis=(0, 1), keepdims=True)


def _dense_sum_kernel(x_ref, out_ref):
    i = pl.program_id(0)
    partial = _fsum_scaled(x_ref[...])

    @pl.when(i == 0)
    def _():
        out_ref[...] = jnp.zeros_like(out_ref)

    out_ref[...] += partial


def _small_sum_kernel(g_ref, out_ref):
    out_ref[...] = _fsum_scaled(g_ref[...])


def _sc_gather_kernel(x_hbm, lbl_hbm, out_hbm, lbl_v, rows_v, g_v):
    wid = lax.axis_index("s") * _NC + lax.axis_index("c")
    base = wid * _PER_W
    pltpu.sync_copy(lbl_hbm.at[pl.ds(base, _PER_W)], lbl_v)

    iota = lax.iota(jnp.int32, _L)
    for chunk in range(_NCHUNK):
        chunk_base = base + chunk * _RCHUNK
        dma_base = jnp.minimum(chunk_base, N_ROWS - _RCHUNK)
        pltpu.sync_copy(x_hbm.at[pl.ds(dma_base, _RCHUNK)], rows_v)
        off = chunk_base - dma_base

        def body(i, _, chunk=chunk, off=off):
            l = lbl_v[pl.ds(chunk * _RCHUNK + i * _L, _L)]
            local_raw = off + i * _L + iota
            valid = (l < N_COLS) & (local_raw < _RCHUNK)
            l_safe = jnp.where(valid, l, 0)
            local = jnp.minimum(local_raw, _RCHUNK - 1)
            got = plsc.load_gather(rows_v, [local, l_safe])
            g_v[pl.ds(chunk * _RCHUNK + i * _L, _L)] = jnp.where(
                valid, got, NEG_FILL
            )
            return _

        lax.fori_loop(0, _RCHUNK // _L, body, None, unroll=5)

    pltpu.sync_copy(g_v, out_hbm.at[wid])


def kernel(cls_pred, cls_label):
    N, C = cls_pred.shape

    s1 = pl.pallas_call(
        _dense_sum_kernel,
        grid=(10,),
        in_specs=[pl.BlockSpec((N // 10, C), lambda i: (i, 0))],
        out_specs=pl.BlockSpec((1, 1), lambda i: (0, 0)),
        out_shape=jax.ShapeDtypeStruct((1, 1), jnp.float32),
    )(cls_pred)

    lbl = cls_label.astype(jnp.int32)
    lblp = jnp.concatenate([lbl, jnp.full((_NPAD - N,), C, jnp.int32)])

    mesh = plsc.VectorSubcoreMesh(core_axis_name="c", subcore_axis_name="s")
    g = pl.kernel(
        _sc_gather_kernel,
        mesh=mesh,
        out_type=jax.ShapeDtypeStruct((_NW, _PER_W), jnp.float32),
        scratch_types=[
            pltpu.VMEM((_PER_W,), jnp.int32),
            pltpu.VMEM((_RCHUNK, N_COLS), jnp.float32),
            pltpu.VMEM((_PER_W,), jnp.float32),
        ],
        compiler_params=pltpu.CompilerParams(
            use_tc_tiling_on_sc=True, needs_layout_passes=False
        ),
    )(cls_pred, lblp)

    s2 = pl.pallas_call(
        _small_sum_kernel,
        grid=(1,),
        in_specs=[pl.BlockSpec(g.shape, lambda i: (0, 0))],
        out_specs=pl.BlockSpec((1, 1), lambda i: (0, 0)),
        out_shape=jax.ShapeDtypeStruct((1, 1), jnp.float32),
    )(g)

    return (-_LN2 * LOSS_WEIGHT / (N * C)) * (0.75 * s1[0, 0] - 0.5 * s2[0, 0])
